# baseline (speedup 1.0000x reference)
import functools

import jax
import jax.numpy as jnp
from jax import lax
from jax.experimental import pallas as pl
from jax.experimental.pallas import tpu as pltpu
from jax.experimental.pallas import tpu_sc as plsc

N_NODES = 10000
N_EDGES = 320000
D_FEAT = 128
D_EDGE = 16

NC = 2
NS = 16
NW = NC * NS
G = 128
N_CHUNKS = N_EDGES // G
NCW = N_CHUNKS // NW
N_EXTRA = N_CHUNKS % NW
CHUNK_F32 = G * D_EDGE


def _tc_tables(x_ref, w_ref, b_ref, p_ref, q_ref):
    res = jnp.dot(x_ref[...], w_ref[...], preferred_element_type=jnp.float32)
    p_ref[...] = res[:, :D_EDGE]
    q_ref[...] = res[:, D_EDGE:] + b_ref[...]


_mesh = plsc.VectorSubcoreMesh(core_axis_name="c", subcore_axis_name="s")


@functools.partial(
    pl.kernel,
    mesh=_mesh,
    compiler_params=pltpu.CompilerParams(use_tc_tiling_on_sc=False,
                                         needs_layout_passes=False),
    out_type=jax.ShapeDtypeStruct((N_EDGES * D_EDGE,), jnp.float32),
    scratch_types=[
        pltpu.VMEM((NCW * G,), jnp.int32),
        pltpu.VMEM((NCW * G,), jnp.int32),
        pltpu.VMEM((2, G, D_EDGE), jnp.float32),
        pltpu.VMEM((2, G, D_EDGE), jnp.float32),
        pltpu.VMEM((CHUNK_F32,), jnp.float32),
        pltpu.VMEM((CHUNK_F32,), jnp.float32),
        pltpu.SemaphoreType.DMA,
        pltpu.SemaphoreType.DMA,
        pltpu.SemaphoreType.DMA,
        pltpu.SemaphoreType.DMA,
        pltpu.SemaphoreType.DMA,
    ],
)
def _sc_gather_add(p_hbm, q_hbm, ei_hbm, out_hbm,
                   sidx, didx, pbuf, qbuf, obuf_a, obuf_b,
                   sem_i, sem_ga, sem_gb, sem_oa, sem_ob):
    wid = lax.axis_index("s") * NC + lax.axis_index("c")
    start_c = NCW * wid + jnp.minimum(wid, N_EXTRA)
    base_e = start_c * G
    sem_g = (sem_ga, sem_gb)
    sem_o = (sem_oa, sem_ob)
    obuf = (obuf_a, obuf_b)
    iot = lax.iota(jnp.int32, 16) * G

    ci0 = pltpu.async_copy(ei_hbm.at[0, pl.ds(base_e, NCW * G)], sidx, sem_i)
    ci1 = pltpu.async_copy(ei_hbm.at[1, pl.ds(base_e, NCW * G)], didx, sem_i)
    ci0.wait()
    ci1.wait()

    def fire_gathers(g, s):
        pltpu.async_copy(p_hbm.at[sidx.at[pl.ds(g * G, G)]],
                         pbuf.at[s], sem_g[s])
        pltpu.async_copy(q_hbm.at[didx.at[pl.ds(g * G, G)]],
                         qbuf.at[s], sem_g[s])

    def wait_gathers(s):
        pltpu.make_async_copy(p_hbm.at[sidx.at[pl.ds(0, G)]],
                              pbuf.at[s], sem_g[s]).wait()
        pltpu.make_async_copy(q_hbm.at[didx.at[pl.ds(0, G)]],
                              qbuf.at[s], sem_g[s]).wait()

    def add_rows(s):
        def row(i, c):
            val = pbuf[s, i, :] + qbuf[s, i, :]
            plsc.store_scatter(obuf[s], [iot + i], val)
            return c
        lax.fori_loop(0, G, row, 0, unroll=8)

    def fire_store(g, s):
        pltpu.async_copy(obuf[s],
                         out_hbm.at[pl.ds((start_c + g) * CHUNK_F32, CHUNK_F32)],
                         sem_o[s])

    def wait_store(s):
        pltpu.make_async_copy(obuf[s],
                              out_hbm.at[pl.ds(0, CHUNK_F32)], sem_o[s]).wait()

    def pair(m, first, fire_next):
        g0 = 2 * m
        fire_gathers(g0 + 1, 1)
        wait_gathers(0)
        if not first:
            wait_store(0)
        add_rows(0)
        fire_store(g0, 0)
        if fire_next:
            fire_gathers(g0 + 2, 0)
        wait_gathers(1)
        if not first:
            wait_store(1)
        add_rows(1)
        fire_store(g0 + 1, 1)

    fire_gathers(0, 0)
    pair(0, first=True, fire_next=True)

    def body(m, c):
        pair(m, first=False, fire_next=True)
        return c

    lax.fori_loop(1, NCW // 2 - 1, body, 0)
    pair(NCW // 2 - 1, first=False, fire_next=False)
    wait_store(0)
    wait_store(1)

    @pl.when(wid < N_EXTRA)
    def _extra():
        ec = start_c + NCW
        pltpu.sync_copy(ei_hbm.at[0, pl.ds(ec * G, G)], sidx.at[pl.ds(0, G)])
        pltpu.sync_copy(ei_hbm.at[1, pl.ds(ec * G, G)], didx.at[pl.ds(0, G)])
        fire_gathers(0, 0)
        wait_gathers(0)
        add_rows(0)
        pltpu.async_copy(obuf[0],
                         out_hbm.at[pl.ds(ec * CHUNK_F32, CHUNK_F32)], sem_oa)
        wait_store(0)


_TC_GRID = 50
_CB = N_CHUNKS // _TC_GRID


def _tc_relayout(in_ref, out_ref):
    x = in_ref[...]
    out_ref[...] = jnp.swapaxes(x, 1, 2).reshape(_CB * G, D_EDGE)


def kernel(x, edge_index, pos, W, b):
    wcat = jnp.concatenate([W[:D_FEAT, :], W[D_FEAT:, :]], axis=1)
    p, q = pl.pallas_call(
        _tc_tables,
        out_shape=[
            jax.ShapeDtypeStruct((N_NODES, D_EDGE), jnp.float32),
            jax.ShapeDtypeStruct((N_NODES, D_EDGE), jnp.float32),
        ],
    )(x, wcat, b.reshape(1, D_EDGE))
    flat = _sc_gather_add(p, q, edge_index)
    t = flat.reshape(N_CHUNKS, D_EDGE, G)
    return pl.pallas_call(
        _tc_relayout,
        grid=(_TC_GRID,),
        in_specs=[pl.BlockSpec((_CB, D_EDGE, G), lambda i: (i, 0, 0))],
        out_specs=pl.BlockSpec((_CB * G, D_EDGE), lambda i: (i, 0)),
        out_shape=jax.ShapeDtypeStruct((N_EDGES, D_EDGE), jnp.float32),
    )(t)


# speedup vs baseline: 2.0154x; 2.0154x over previous
import functools

import jax
import jax.numpy as jnp
from jax import lax
from jax.experimental import pallas as pl
from jax.experimental.pallas import tpu as pltpu
from jax.experimental.pallas import tpu_sc as plsc

N_NODES = 10000
N_EDGES = 320000
D_FEAT = 128
D_EDGE = 16

NC = 2
NS = 16
NW = NC * NS
G = 128
N_CHUNKS = N_EDGES // G
NCW = N_CHUNKS // NW
N_EXTRA = N_CHUNKS % NW
CHUNK_F32 = G * D_EDGE
HALF = CHUNK_F32 // 2
BAND_F32 = N_CHUNKS * 8 * G


def _tc_tables(x_ref, w_ref, b_ref, p_ref, q_ref):
    res = jnp.dot(x_ref[...], w_ref[...], preferred_element_type=jnp.float32)
    p_ref[...] = res[:, :D_EDGE]
    q_ref[...] = res[:, D_EDGE:] + b_ref[...]


_mesh = plsc.VectorSubcoreMesh(core_axis_name="c", subcore_axis_name="s")


@functools.partial(
    pl.kernel,
    mesh=_mesh,
    compiler_params=pltpu.CompilerParams(use_tc_tiling_on_sc=False,
                                         needs_layout_passes=False),
    out_type=jax.ShapeDtypeStruct((N_EDGES * D_EDGE,), jnp.float32),
    scratch_types=[
        pltpu.VMEM((NCW * G,), jnp.int32),
        pltpu.VMEM((NCW * G,), jnp.int32),
        pltpu.VMEM((2, G, D_EDGE), jnp.float32),
        pltpu.VMEM((2, G, D_EDGE), jnp.float32),
        pltpu.VMEM((CHUNK_F32,), jnp.float32),
        pltpu.VMEM((CHUNK_F32,), jnp.float32),
        pltpu.SemaphoreType.DMA,
        pltpu.SemaphoreType.DMA,
        pltpu.SemaphoreType.DMA,
        pltpu.SemaphoreType.DMA,
        pltpu.SemaphoreType.DMA,
    ],
)
def _sc_gather_add(p_hbm, q_hbm, ei_hbm, out_hbm,
                   sidx, didx, pbuf, qbuf, obuf_a, obuf_b,
                   sem_i, sem_ga, sem_gb, sem_oa, sem_ob):
    wid = lax.axis_index("s") * NC + lax.axis_index("c")
    start_c = NCW * wid + jnp.minimum(wid, N_EXTRA)
    base_e = start_c * G
    sem_g = (sem_ga, sem_gb)
    sem_o = (sem_oa, sem_ob)
    obuf = (obuf_a, obuf_b)
    iot = lax.iota(jnp.int32, 16) * G

    ci0 = pltpu.async_copy(ei_hbm.at[0, pl.ds(base_e, NCW * G)], sidx, sem_i)
    ci1 = pltpu.async_copy(ei_hbm.at[1, pl.ds(base_e, NCW * G)], didx, sem_i)
    ci0.wait()
    ci1.wait()

    def fire_gathers(g, s):
        pltpu.async_copy(p_hbm.at[sidx.at[pl.ds(g * G, G)]],
                         pbuf.at[s], sem_g[s])
        pltpu.async_copy(q_hbm.at[didx.at[pl.ds(g * G, G)]],
                         qbuf.at[s], sem_g[s])

    def wait_gathers(s):
        pltpu.make_async_copy(p_hbm.at[sidx.at[pl.ds(0, G)]],
                              pbuf.at[s], sem_g[s]).wait()
        pltpu.make_async_copy(q_hbm.at[didx.at[pl.ds(0, G)]],
                              qbuf.at[s], sem_g[s]).wait()

    def add_rows(s):
        def row(i, c):
            val = pbuf[s, i, :] + qbuf[s, i, :]
            plsc.store_scatter(obuf[s], [iot + i], val)
            return c
        lax.fori_loop(0, G, row, 0, unroll=8)

    def fire_store_at(c, s, sem):
        pltpu.async_copy(obuf[s].at[pl.ds(0, HALF)],
                         out_hbm.at[pl.ds(c * HALF, HALF)], sem)
        pltpu.async_copy(obuf[s].at[pl.ds(HALF, HALF)],
                         out_hbm.at[pl.ds(BAND_F32 + c * HALF, HALF)], sem)

    def fire_store(g, s):
        fire_store_at(start_c + g, s, sem_o[s])

    def wait_store(s):
        pltpu.make_async_copy(obuf[s].at[pl.ds(0, HALF)],
                              out_hbm.at[pl.ds(0, HALF)], sem_o[s]).wait()
        pltpu.make_async_copy(obuf[s].at[pl.ds(0, HALF)],
                              out_hbm.at[pl.ds(0, HALF)], sem_o[s]).wait()

    def pair(m, first, fire_next):
        g0 = 2 * m
        fire_gathers(g0 + 1, 1)
        wait_gathers(0)
        if not first:
            wait_store(0)
        add_rows(0)
        fire_store(g0, 0)
        if fire_next:
            fire_gathers(g0 + 2, 0)
        wait_gathers(1)
        if not first:
            wait_store(1)
        add_rows(1)
        fire_store(g0 + 1, 1)

    fire_gathers(0, 0)
    pair(0, first=True, fire_next=True)

    def body(m, c):
        pair(m, first=False, fire_next=True)
        return c

    lax.fori_loop(1, NCW // 2 - 1, body, 0)
    pair(NCW // 2 - 1, first=False, fire_next=False)
    wait_store(0)
    wait_store(1)

    @pl.when(wid < N_EXTRA)
    def _extra():
        ec = start_c + NCW
        pltpu.sync_copy(ei_hbm.at[0, pl.ds(ec * G, G)], sidx.at[pl.ds(0, G)])
        pltpu.sync_copy(ei_hbm.at[1, pl.ds(ec * G, G)], didx.at[pl.ds(0, G)])
        fire_gathers(0, 0)
        wait_gathers(0)
        add_rows(0)
        fire_store_at(ec, 0, sem_oa)
        wait_store(0)


def kernel(x, edge_index, pos, W, b):
    wcat = jnp.concatenate([W[:D_FEAT, :], W[D_FEAT:, :]], axis=1)
    p, q = pl.pallas_call(
        _tc_tables,
        out_shape=[
            jax.ShapeDtypeStruct((N_NODES, D_EDGE), jnp.float32),
            jax.ShapeDtypeStruct((N_NODES, D_EDGE), jnp.float32),
        ],
    )(x, wcat, b.reshape(1, D_EDGE))
    flat = _sc_gather_add(p, q, edge_index)
    arr = flat.reshape(2, N_CHUNKS, 8, G)
    return arr.transpose(1, 3, 0, 2).reshape(N_EDGES, D_EDGE)


# speedup vs baseline: 2.5854x; 1.2828x over previous
import functools

import jax
import jax.numpy as jnp
from jax import lax
from jax.experimental import pallas as pl
from jax.experimental.pallas import tpu as pltpu
from jax.experimental.pallas import tpu_sc as plsc

N_NODES = 10000
N_EDGES = 320000
D_FEAT = 128
D_EDGE = 16

NC = 2
NS = 16
NW = NC * NS
G = 128
N_CHUNKS = N_EDGES // G
NCW = N_CHUNKS // NW
N_EXTRA = N_CHUNKS % NW
CHUNK_F32 = G * D_EDGE
HALF = CHUNK_F32 // 2
BAND_F32 = N_CHUNKS * 8 * G


def _tc_tables(x_ref, w_ref, b_ref, p_ref, q_ref):
    res = jnp.dot(x_ref[...], w_ref[...], preferred_element_type=jnp.float32)
    p_ref[...] = res[:, :D_EDGE]
    q_ref[...] = res[:, D_EDGE:] + b_ref[...]


_mesh = plsc.VectorSubcoreMesh(core_axis_name="c", subcore_axis_name="s")


@functools.partial(
    pl.kernel,
    mesh=_mesh,
    compiler_params=pltpu.CompilerParams(use_tc_tiling_on_sc=False,
                                         needs_layout_passes=False),
    out_type=jax.ShapeDtypeStruct((N_EDGES * D_EDGE,), jnp.float32),
    scratch_types=[
        pltpu.VMEM((NCW * G,), jnp.int32),
        pltpu.VMEM((NCW * G,), jnp.int32),
        pltpu.VMEM((2, G, D_EDGE), jnp.float32),
        pltpu.VMEM((2, G, D_EDGE), jnp.float32),
        pltpu.VMEM((CHUNK_F32,), jnp.float32),
        pltpu.VMEM((CHUNK_F32,), jnp.float32),
        pltpu.SemaphoreType.DMA,
        pltpu.SemaphoreType.DMA,
        pltpu.SemaphoreType.DMA,
        pltpu.SemaphoreType.DMA,
        pltpu.SemaphoreType.DMA,
    ],
)
def _sc_gather_add(p_hbm, q_hbm, ei_hbm, out_hbm,
                   sidx, didx, pbuf, qbuf, obuf_a, obuf_b,
                   sem_i, sem_ga, sem_gb, sem_oa, sem_ob):
    wid = lax.axis_index("s") * NC + lax.axis_index("c")
    start_c = NCW * wid + jnp.minimum(wid, N_EXTRA)
    base_e = start_c * G
    sem_g = (sem_ga, sem_gb)
    sem_o = (sem_oa, sem_ob)
    obuf = (obuf_a, obuf_b)
    iot = lax.iota(jnp.int32, 16) * G

    ci0 = pltpu.async_copy(ei_hbm.at[0, pl.ds(base_e, NCW * G)], sidx, sem_i)
    ci1 = pltpu.async_copy(ei_hbm.at[1, pl.ds(base_e, NCW * G)], didx, sem_i)
    ci0.wait()
    ci1.wait()

    def fire_gathers(g, s):
        pltpu.async_copy(p_hbm.at[sidx.at[pl.ds(g * G, G)]],
                         pbuf.at[s], sem_g[s])
        pltpu.async_copy(q_hbm.at[didx.at[pl.ds(g * G, G)]],
                         qbuf.at[s], sem_g[s])

    def wait_gathers(s):
        pltpu.make_async_copy(p_hbm.at[sidx.at[pl.ds(0, G)]],
                              pbuf.at[s], sem_g[s]).wait()
        pltpu.make_async_copy(q_hbm.at[didx.at[pl.ds(0, G)]],
                              qbuf.at[s], sem_g[s]).wait()

    def add_rows(s):
        @plsc.parallel_loop(0, G, unroll=8)
        def row(i):
            val = pbuf[s, i, :] + qbuf[s, i, :]
            plsc.store_scatter(obuf[s], [iot + i], val)

    def fire_store_at(c, s, sem):
        pltpu.async_copy(obuf[s].at[pl.ds(0, HALF)],
                         out_hbm.at[pl.ds(c * HALF, HALF)], sem)
        pltpu.async_copy(obuf[s].at[pl.ds(HALF, HALF)],
                         out_hbm.at[pl.ds(BAND_F32 + c * HALF, HALF)], sem)

    def fire_store(g, s):
        fire_store_at(start_c + g, s, sem_o[s])

    def wait_store(s):
        pltpu.make_async_copy(obuf[s].at[pl.ds(0, HALF)],
                              out_hbm.at[pl.ds(0, HALF)], sem_o[s]).wait()
        pltpu.make_async_copy(obuf[s].at[pl.ds(0, HALF)],
                              out_hbm.at[pl.ds(0, HALF)], sem_o[s]).wait()

    def pair(m, first, fire_next):
        g0 = 2 * m
        fire_gathers(g0 + 1, 1)
        wait_gathers(0)
        if not first:
            wait_store(0)
        add_rows(0)
        fire_store(g0, 0)
        if fire_next:
            fire_gathers(g0 + 2, 0)
        wait_gathers(1)
        if not first:
            wait_store(1)
        add_rows(1)
        fire_store(g0 + 1, 1)

    fire_gathers(0, 0)
    pair(0, first=True, fire_next=True)

    def body(m, c):
        pair(m, first=False, fire_next=True)
        return c

    lax.fori_loop(1, NCW // 2 - 1, body, 0)
    pair(NCW // 2 - 1, first=False, fire_next=False)
    wait_store(0)
    wait_store(1)

    @pl.when(wid < N_EXTRA)
    def _extra():
        ec = start_c + NCW
        pltpu.sync_copy(ei_hbm.at[0, pl.ds(ec * G, G)], sidx.at[pl.ds(0, G)])
        pltpu.sync_copy(ei_hbm.at[1, pl.ds(ec * G, G)], didx.at[pl.ds(0, G)])
        fire_gathers(0, 0)
        wait_gathers(0)
        add_rows(0)
        fire_store_at(ec, 0, sem_oa)
        wait_store(0)


def kernel(x, edge_index, pos, W, b):
    wcat = jnp.concatenate([W[:D_FEAT, :], W[D_FEAT:, :]], axis=1)
    p, q = pl.pallas_call(
        _tc_tables,
        out_shape=[
            jax.ShapeDtypeStruct((N_NODES, D_EDGE), jnp.float32),
            jax.ShapeDtypeStruct((N_NODES, D_EDGE), jnp.float32),
        ],
    )(x, wcat, b.reshape(1, D_EDGE))
    flat = _sc_gather_add(p, q, edge_index)
    arr = flat.reshape(2, N_CHUNKS, 8, G)
    return arr.transpose(1, 3, 0, 2).reshape(N_EDGES, D_EDGE)


# speedup vs baseline: 2.8122x; 1.0877x over previous
import functools

import jax
import jax.numpy as jnp
from jax import lax
from jax.experimental import pallas as pl
from jax.experimental.pallas import tpu as pltpu
from jax.experimental.pallas import tpu_sc as plsc

N_NODES = 10000
N_EDGES = 320000
D_FEAT = 128
D_EDGE = 16

NC = 2
NS = 16
NW = NC * NS
G = 128
N_CHUNKS = N_EDGES // G
NCW = N_CHUNKS // NW
N_EXTRA = N_CHUNKS % NW
CHUNK_F32 = G * D_EDGE
HALF = CHUNK_F32 // 2
BAND_F32 = N_CHUNKS * 8 * G
NSLOT = 3
NTRI = NCW // NSLOT


def _tc_tables(x_ref, w_ref, b_ref, p_ref, q_ref):
    res = jnp.dot(x_ref[...], w_ref[...], preferred_element_type=jnp.float32)
    p_ref[...] = res[:, :D_EDGE]
    q_ref[...] = res[:, D_EDGE:] + b_ref[...]


_mesh = plsc.VectorSubcoreMesh(core_axis_name="c", subcore_axis_name="s")


@functools.partial(
    pl.kernel,
    mesh=_mesh,
    compiler_params=pltpu.CompilerParams(use_tc_tiling_on_sc=False,
                                         needs_layout_passes=False),
    out_type=jax.ShapeDtypeStruct((N_EDGES * D_EDGE,), jnp.float32),
    scratch_types=[
        pltpu.VMEM((NCW, 2, G), jnp.int32),
        pltpu.VMEM((NSLOT, G, D_EDGE), jnp.float32),
        pltpu.VMEM((NSLOT, G, D_EDGE), jnp.float32),
        pltpu.VMEM((CHUNK_F32,), jnp.float32),
        pltpu.VMEM((CHUNK_F32,), jnp.float32),
        pltpu.VMEM((CHUNK_F32,), jnp.float32),
        pltpu.SemaphoreType.DMA,
        pltpu.SemaphoreType.DMA,
        pltpu.SemaphoreType.DMA,
        pltpu.SemaphoreType.DMA,
        pltpu.SemaphoreType.DMA,
        pltpu.SemaphoreType.DMA,
        pltpu.SemaphoreType.DMA,
    ],
)
def _sc_gather_add(p_hbm, q_hbm, ei_hbm, out_hbm,
                   eibuf, pbuf, qbuf, obuf0, obuf1, obuf2,
                   sem_i, sem_g0, sem_g1, sem_g2, sem_o0, sem_o1, sem_o2):
    wid = lax.axis_index("s") * NC + lax.axis_index("c")
    start_c = NCW * wid + jnp.minimum(wid, N_EXTRA)
    sem_g = (sem_g0, sem_g1, sem_g2)
    sem_o = (sem_o0, sem_o1, sem_o2)
    obuf = (obuf0, obuf1, obuf2)
    iot = lax.iota(jnp.int32, 16) * G

    pltpu.async_copy(ei_hbm.at[pl.ds(start_c, NCW)], eibuf, sem_i).wait()

    def fire_gathers(g, s):
        pltpu.async_copy(p_hbm.at[eibuf.at[g, 0]], pbuf.at[s], sem_g[s])
        pltpu.async_copy(q_hbm.at[eibuf.at[g, 1]], qbuf.at[s], sem_g[s])

    def wait_gathers(s):
        pltpu.make_async_copy(p_hbm.at[eibuf.at[0, 0]],
                              pbuf.at[s], sem_g[s]).wait()
        pltpu.make_async_copy(q_hbm.at[eibuf.at[0, 1]],
                              qbuf.at[s], sem_g[s]).wait()

    def add_rows(s):
        @plsc.parallel_loop(0, G, unroll=8)
        def row(i):
            val = pbuf[s, i, :] + qbuf[s, i, :]
            plsc.store_scatter(obuf[s], [iot + i], val)

    def fire_store_at(c, s, sem):
        pltpu.async_copy(obuf[s].at[pl.ds(0, HALF)],
                         out_hbm.at[pl.ds(c * HALF, HALF)], sem)
        pltpu.async_copy(obuf[s].at[pl.ds(HALF, HALF)],
                         out_hbm.at[pl.ds(BAND_F32 + c * HALF, HALF)], sem)

    def wait_store(s):
        pltpu.make_async_copy(obuf[s].at[pl.ds(0, HALF)],
                              out_hbm.at[pl.ds(0, HALF)], sem_o[s]).wait()
        pltpu.make_async_copy(obuf[s].at[pl.ds(0, HALF)],
                              out_hbm.at[pl.ds(0, HALF)], sem_o[s]).wait()

    def chunk(c, s, first, fire_next):
        wait_gathers(s)
        if not first:
            wait_store(s)
        add_rows(s)
        fire_store_at(start_c + c, s, sem_o[s])
        if fire_next:
            fire_gathers(c + NSLOT, s)

    for s in range(NSLOT):
        fire_gathers(s, s)
    for j in range(NSLOT):
        chunk(j, j, first=True, fire_next=True)

    def body(m, cy):
        for j in range(NSLOT):
            chunk(3 * m + j, j, first=False, fire_next=True)
        return cy

    lax.fori_loop(1, NTRI - 1, body, 0)
    for j in range(NSLOT):
        chunk(3 * (NTRI - 1) + j, j, first=False, fire_next=False)
    for s in range(NSLOT):
        wait_store(s)

    @pl.when(wid < N_EXTRA)
    def _extra():
        ec = start_c + NCW
        pltpu.sync_copy(ei_hbm.at[pl.ds(ec, 1)], eibuf.at[pl.ds(0, 1)])
        fire_gathers(0, 0)
        wait_gathers(0)
        add_rows(0)
        fire_store_at(ec, 0, sem_o0)
        wait_store(0)


def kernel(x, edge_index, pos, W, b):
    wcat = jnp.concatenate([W[:D_FEAT, :], W[D_FEAT:, :]], axis=1)
    p, q = pl.pallas_call(
        _tc_tables,
        out_shape=[
            jax.ShapeDtypeStruct((N_NODES, D_EDGE), jnp.float32),
            jax.ShapeDtypeStruct((N_NODES, D_EDGE), jnp.float32),
        ],
    )(x, wcat, b.reshape(1, D_EDGE))
    ei3 = edge_index.reshape(2, N_CHUNKS, G).transpose(1, 0, 2)
    flat = _sc_gather_add(p, q, ei3)
    arr = flat.reshape(2, N_CHUNKS, 8, G)
    return arr.transpose(1, 3, 0, 2).reshape(N_EDGES, D_EDGE)
